# baseline (speedup 1.0000x reference)
import functools

import numpy as np

import jax
import jax.numpy as jnp
from jax import lax
from jax.experimental import pallas as pl
from jax.experimental.pallas import tpu as pltpu
from jax.experimental.pallas import tpu_sc as plsc

_A = 9
_H = 50
_W = 76
_N = _H * _W * _A
_ROWS = 272
_NP = _ROWS * 128
_PRE_NMS = 6000
_POST_NMS = 300
_OUT_ROWS = 304
_NMS_T = 0.7
_FEAT_STRIDE = 16.0
_MIN_SIZE = 3.0
_IMIN = -2147483648

_NTILES = 32
_CHUNK = _NP // _NTILES
_NV = _CHUNK // 16

_ANCH = (
    (-84.0, -40.0, 99.0, 55.0),
    (-176.0, -88.0, 191.0, 103.0),
    (-360.0, -184.0, 375.0, 199.0),
    (-56.0, -56.0, 71.0, 71.0),
    (-120.0, -120.0, 135.0, 135.0),
    (-248.0, -248.0, 263.0, 263.0),
    (-36.0, -80.0, 51.0, 95.0),
    (-80.0, -168.0, 95.0, 183.0),
    (-168.0, -344.0, 183.0, 359.0),
)

def _anchor_grid():
    ii = np.arange(_NP)
    aa = ii % _A
    cell = ii // _A
    colf = (cell % _W).astype(np.float32) * _FEAT_STRIDE
    rowf = (cell // _W).astype(np.float32) * _FEAT_STRIDE
    anch = np.asarray(_ANCH, np.float32)[aa]
    return (anch[:, 0] + colf, anch[:, 1] + rowf,
            anch[:, 2] + colf, anch[:, 3] + rowf)

_AG = _anchor_grid()


def _sc_transform_body(sc_hbm, dx_hbm, dy_hbm, dw_hbm, dh_hbm,
                       a1_hbm, a2_hbm, a3_hbm, a4_hbm, im_hbm, out_hbm,
                       sc_v, dx_v, dy_v, dw_v, dh_v,
                       a1_v, a2_v, a3_v, a4_v, im_v,
                       x1_v, y1_v, x2_v, y2_v, ar_v, fk_v):
    wid = lax.axis_index("s") * 2 + lax.axis_index("c")
    base = wid * _CHUNK
    pltpu.sync_copy(sc_hbm.at[pl.ds(base, _CHUNK)], sc_v)
    pltpu.sync_copy(dx_hbm.at[pl.ds(base, _CHUNK)], dx_v)
    pltpu.sync_copy(dy_hbm.at[pl.ds(base, _CHUNK)], dy_v)
    pltpu.sync_copy(dw_hbm.at[pl.ds(base, _CHUNK)], dw_v)
    pltpu.sync_copy(dh_hbm.at[pl.ds(base, _CHUNK)], dh_v)
    pltpu.sync_copy(a1_hbm.at[pl.ds(base, _CHUNK)], a1_v)
    pltpu.sync_copy(a2_hbm.at[pl.ds(base, _CHUNK)], a2_v)
    pltpu.sync_copy(a3_hbm.at[pl.ds(base, _CHUNK)], a3_v)
    pltpu.sync_copy(a4_hbm.at[pl.ds(base, _CHUNK)], a4_v)
    pltpu.sync_copy(im_hbm, im_v)

    im_h = im_v[pl.ds(0, 16)]
    im_w = im_v[pl.ds(16, 16)]
    min_sz = im_v[pl.ds(32, 16)]
    lane = lax.iota(jnp.int32, 16)

    def step(j, _):
        off = j * 16
        idx = base + off + lane
        ax1 = a1_v[pl.ds(off, 16)]
        ay1 = a2_v[pl.ds(off, 16)]
        ax2 = a3_v[pl.ds(off, 16)]
        ay2 = a4_v[pl.ds(off, 16)]
        w = ax2 - ax1 + 1.0
        h = ay2 - ay1 + 1.0
        cx = ax1 + 0.5 * w
        cy = ay1 + 0.5 * h
        pcx = dx_v[pl.ds(off, 16)] * w + cx
        pcy = dy_v[pl.ds(off, 16)] * h + cy
        pw = jnp.exp(dw_v[pl.ds(off, 16)]) * w
        ph = jnp.exp(dh_v[pl.ds(off, 16)]) * h
        x1 = jnp.clip(pcx - 0.5 * pw, 0.0, im_w - 1.0)
        y1 = jnp.clip(pcy - 0.5 * ph, 0.0, im_h - 1.0)
        x2 = jnp.clip(pcx + 0.5 * pw, 0.0, im_w - 1.0)
        y2 = jnp.clip(pcy + 0.5 * ph, 0.0, im_h - 1.0)
        ws = x2 - x1 + 1.0
        hs = y2 - y1 + 1.0
        valid = (ws >= min_sz) & (hs >= min_sz) & (idx < _N)
        sbits = lax.bitcast_convert_type(sc_v[pl.ds(off, 16)], jnp.int32)
        fk = (jnp.where(valid, sbits + 1, 0) ^ _IMIN)
        x1_v[pl.ds(off, 16)] = x1
        y1_v[pl.ds(off, 16)] = y1
        x2_v[pl.ds(off, 16)] = x2
        y2_v[pl.ds(off, 16)] = y2
        ar_v[pl.ds(off, 16)] = ws * hs
        fk_v[pl.ds(off, 16)] = lax.bitcast_convert_type(fk, jnp.float32)
        return 0

    lax.fori_loop(0, _NV, step, 0)
    pltpu.sync_copy(x1_v, out_hbm.at[pl.ds(0 * _NP + base, _CHUNK)])
    pltpu.sync_copy(y1_v, out_hbm.at[pl.ds(1 * _NP + base, _CHUNK)])
    pltpu.sync_copy(x2_v, out_hbm.at[pl.ds(2 * _NP + base, _CHUNK)])
    pltpu.sync_copy(y2_v, out_hbm.at[pl.ds(3 * _NP + base, _CHUNK)])
    pltpu.sync_copy(ar_v, out_hbm.at[pl.ds(4 * _NP + base, _CHUNK)])
    pltpu.sync_copy(fk_v, out_hbm.at[pl.ds(5 * _NP + base, _CHUNK)])


_sc_transform = functools.partial(
    pl.kernel,
    mesh=plsc.VectorSubcoreMesh(core_axis_name="c", subcore_axis_name="s"),
    out_type=jax.ShapeDtypeStruct((6 * _NP,), jnp.float32),
    scratch_types=[pltpu.VMEM((_CHUNK,), jnp.float32)] * 9
    + [pltpu.VMEM((48,), jnp.float32)]
    + [pltpu.VMEM((_CHUNK,), jnp.float32)] * 6,
)(_sc_transform_body)


def _tc_body(d_r, out_ref, sl_r):
    x1_r = d_r.at[0]
    y1_r = d_r.at[1]
    x2_r = d_r.at[2]
    y2_r = d_r.at[3]
    ar_r = d_r.at[4]
    i2 = (lax.broadcasted_iota(jnp.int32, (_ROWS, 128), 0) * 128
          + lax.broadcasted_iota(jnp.int32, (_ROWS, 128), 1))
    imin = jnp.int32(_IMIN)
    fkey = d_r[5].view(jnp.int32)

    def bs_bit(b, cu):
        cand = cu | lax.shift_left(jnp.int32(1), 31 - b)
        cnt = jnp.sum((fkey >= (cand ^ imin)).astype(jnp.int32))
        return jnp.where(cnt >= _PRE_NMS, cand, cu)

    t_bits = lax.fori_loop(0, 32, bs_bit, jnp.int32(0))
    ft = t_bits ^ imin
    slots = _PRE_NMS - jnp.sum((fkey > ft).astype(jnp.int32))
    eqm = (fkey == ft).astype(jnp.int32)

    def bs_idx(b, c):
        cand = c | lax.shift_left(jnp.int32(1), 16 - b)
        cnt = jnp.sum(jnp.where(i2 < cand, eqm, 0))
        return jnp.where(cnt <= slots, cand, c)

    c = lax.fori_loop(1, 17, bs_idx, jnp.int32(0))
    elig = (fkey > ft) | ((t_bits != 0) & (fkey == ft) & (i2 < c))
    sl_r[...] = jnp.where(elig, fkey, imin)

    lane = lax.broadcasted_iota(jnp.int32, (1, 128), 1)

    def nms_step(k, m):
        sl = sl_r[...]
        found = m > imin
        pick = jnp.min(jnp.where(sl == m, i2, _NP))
        pick = jnp.where(pick == _NP, 0, pick)
        pr = pick // 128
        pc = lax.rem(pick, 128)
        lm = lane == pc
        zero = jnp.float32(0.0)
        px1 = jnp.sum(jnp.where(lm, x1_r[pl.ds(pr, 1), :], zero))
        py1 = jnp.sum(jnp.where(lm, y1_r[pl.ds(pr, 1), :], zero))
        px2 = jnp.sum(jnp.where(lm, x2_r[pl.ds(pr, 1), :], zero))
        py2 = jnp.sum(jnp.where(lm, y2_r[pl.ds(pr, 1), :], zero))
        pa = jnp.sum(jnp.where(lm, ar_r[pl.ds(pr, 1), :], zero))
        ww = jnp.maximum(0.0, jnp.minimum(px2, x2_r[...])
                         - jnp.maximum(px1, x1_r[...]) + 1.0)
        hh = jnp.maximum(0.0, jnp.minimum(py2, y2_r[...])
                         - jnp.maximum(py1, y1_r[...]) + 1.0)
        inter = ww * hh
        ovr = inter / (pa + ar_r[...] - inter)
        nsl = jnp.where(found & (ovr > _NMS_T), imin, sl)
        sl_r[...] = nsl
        row = jnp.zeros((1, 128), jnp.float32)
        row = jnp.where(lane == 1, px1, row)
        row = jnp.where(lane == 2, py1, row)
        row = jnp.where(lane == 3, px2, row)
        row = jnp.where(lane == 4, py2, row)
        prev0 = out_ref[0:1, :]
        row = jnp.where(found | (k == 0), row, prev0)
        out_ref[pl.ds(k, 1), :] = row
        return jnp.max(nsl)

    m0 = jnp.max(sl_r[...])
    lax.fori_loop(0, _POST_NMS, nms_step, m0)


@jax.jit
def kernel(rpn_cls_prob_reshape, rpn_bbox_pred, im_info):
    scores = jnp.transpose(rpn_cls_prob_reshape[:, _A:], (0, 2, 3, 1)).reshape(-1)
    d = jnp.transpose(rpn_bbox_pred, (0, 2, 3, 1)).reshape(-1, 4)
    pad = _NP - _N

    def p1(v):
        return jnp.pad(v, (0, pad))

    imrow = jnp.concatenate([
        jnp.full((16,), im_info[0, 0], jnp.float32),
        jnp.full((16,), im_info[0, 1], jnp.float32),
        jnp.full((16,), _MIN_SIZE * im_info[0, 2], jnp.float32),
    ])
    res = _sc_transform(
        p1(scores), p1(d[:, 0]), p1(d[:, 1]), p1(d[:, 2]), p1(d[:, 3]),
        jnp.asarray(_AG[0]), jnp.asarray(_AG[1]),
        jnp.asarray(_AG[2]), jnp.asarray(_AG[3]), imrow)

    out = pl.pallas_call(
        _tc_body,
        out_shape=jax.ShapeDtypeStruct((_OUT_ROWS, 128), jnp.float32),
        in_specs=[pl.BlockSpec(memory_space=pltpu.VMEM)],
        out_specs=pl.BlockSpec(memory_space=pltpu.VMEM),
        scratch_shapes=[pltpu.VMEM((_ROWS, 128), jnp.int32)],
    )(res.reshape(6, _ROWS, 128))
    return out[:_POST_NMS, :5]


# speedup vs baseline: 1.0833x; 1.0833x over previous
import functools

import numpy as np

import jax
import jax.numpy as jnp
from jax import lax
from jax.experimental import pallas as pl
from jax.experimental.pallas import tpu as pltpu
from jax.experimental.pallas import tpu_sc as plsc

_A = 9
_H = 50
_W = 76
_N = _H * _W * _A
_ROWS = 272
_NP = _ROWS * 128
_PRE_NMS = 6000
_POST_NMS = 300
_OUT_ROWS = 304
_NMS_T = 0.7
_FEAT_STRIDE = 16.0
_MIN_SIZE = 3.0
_IMIN = -2147483648

_NTILES = 32
_CHUNK = _NP // _NTILES
_NV = _CHUNK // 16

_ANCH = (
    (-84.0, -40.0, 99.0, 55.0),
    (-176.0, -88.0, 191.0, 103.0),
    (-360.0, -184.0, 375.0, 199.0),
    (-56.0, -56.0, 71.0, 71.0),
    (-120.0, -120.0, 135.0, 135.0),
    (-248.0, -248.0, 263.0, 263.0),
    (-36.0, -80.0, 51.0, 95.0),
    (-80.0, -168.0, 95.0, 183.0),
    (-168.0, -344.0, 183.0, 359.0),
)

def _anchor_grid():
    ii = np.arange(_NP)
    aa = ii % _A
    cell = ii // _A
    colf = (cell % _W).astype(np.float32) * _FEAT_STRIDE
    rowf = (cell // _W).astype(np.float32) * _FEAT_STRIDE
    anch = np.asarray(_ANCH, np.float32)[aa]
    return (anch[:, 0] + colf, anch[:, 1] + rowf,
            anch[:, 2] + colf, anch[:, 3] + rowf)

_AG = _anchor_grid()


def _sc_transform_body(sc_hbm, dx_hbm, dy_hbm, dw_hbm, dh_hbm,
                       a1_hbm, a2_hbm, a3_hbm, a4_hbm, im_hbm, out_hbm,
                       sc_v, dx_v, dy_v, dw_v, dh_v,
                       a1_v, a2_v, a3_v, a4_v, im_v,
                       x1_v, y1_v, x2_v, y2_v, ar_v, fk_v):
    wid = lax.axis_index("s") * 2 + lax.axis_index("c")
    base = wid * _CHUNK
    pltpu.sync_copy(sc_hbm.at[pl.ds(base, _CHUNK)], sc_v)
    pltpu.sync_copy(dx_hbm.at[pl.ds(base, _CHUNK)], dx_v)
    pltpu.sync_copy(dy_hbm.at[pl.ds(base, _CHUNK)], dy_v)
    pltpu.sync_copy(dw_hbm.at[pl.ds(base, _CHUNK)], dw_v)
    pltpu.sync_copy(dh_hbm.at[pl.ds(base, _CHUNK)], dh_v)
    pltpu.sync_copy(a1_hbm.at[pl.ds(base, _CHUNK)], a1_v)
    pltpu.sync_copy(a2_hbm.at[pl.ds(base, _CHUNK)], a2_v)
    pltpu.sync_copy(a3_hbm.at[pl.ds(base, _CHUNK)], a3_v)
    pltpu.sync_copy(a4_hbm.at[pl.ds(base, _CHUNK)], a4_v)
    pltpu.sync_copy(im_hbm, im_v)

    im_h = im_v[pl.ds(0, 16)]
    im_w = im_v[pl.ds(16, 16)]
    min_sz = im_v[pl.ds(32, 16)]
    lane = lax.iota(jnp.int32, 16)

    def step(j, _):
        off = j * 16
        idx = base + off + lane
        ax1 = a1_v[pl.ds(off, 16)]
        ay1 = a2_v[pl.ds(off, 16)]
        ax2 = a3_v[pl.ds(off, 16)]
        ay2 = a4_v[pl.ds(off, 16)]
        w = ax2 - ax1 + 1.0
        h = ay2 - ay1 + 1.0
        cx = ax1 + 0.5 * w
        cy = ay1 + 0.5 * h
        pcx = dx_v[pl.ds(off, 16)] * w + cx
        pcy = dy_v[pl.ds(off, 16)] * h + cy
        pw = jnp.exp(dw_v[pl.ds(off, 16)]) * w
        ph = jnp.exp(dh_v[pl.ds(off, 16)]) * h
        x1 = jnp.clip(pcx - 0.5 * pw, 0.0, im_w - 1.0)
        y1 = jnp.clip(pcy - 0.5 * ph, 0.0, im_h - 1.0)
        x2 = jnp.clip(pcx + 0.5 * pw, 0.0, im_w - 1.0)
        y2 = jnp.clip(pcy + 0.5 * ph, 0.0, im_h - 1.0)
        ws = x2 - x1 + 1.0
        hs = y2 - y1 + 1.0
        valid = (ws >= min_sz) & (hs >= min_sz) & (idx < _N)
        sbits = lax.bitcast_convert_type(sc_v[pl.ds(off, 16)], jnp.int32)
        fk = (jnp.where(valid, sbits + 1, 0) ^ _IMIN)
        x1_v[pl.ds(off, 16)] = x1
        y1_v[pl.ds(off, 16)] = y1
        x2_v[pl.ds(off, 16)] = x2
        y2_v[pl.ds(off, 16)] = y2
        ar_v[pl.ds(off, 16)] = ws * hs
        fk_v[pl.ds(off, 16)] = lax.bitcast_convert_type(fk, jnp.float32)
        return 0

    lax.fori_loop(0, _NV, step, 0)
    pltpu.sync_copy(x1_v, out_hbm.at[pl.ds(0 * _NP + base, _CHUNK)])
    pltpu.sync_copy(y1_v, out_hbm.at[pl.ds(1 * _NP + base, _CHUNK)])
    pltpu.sync_copy(x2_v, out_hbm.at[pl.ds(2 * _NP + base, _CHUNK)])
    pltpu.sync_copy(y2_v, out_hbm.at[pl.ds(3 * _NP + base, _CHUNK)])
    pltpu.sync_copy(ar_v, out_hbm.at[pl.ds(4 * _NP + base, _CHUNK)])
    pltpu.sync_copy(fk_v, out_hbm.at[pl.ds(5 * _NP + base, _CHUNK)])


_sc_transform = functools.partial(
    pl.kernel,
    mesh=plsc.VectorSubcoreMesh(core_axis_name="c", subcore_axis_name="s"),
    out_type=jax.ShapeDtypeStruct((6 * _NP,), jnp.float32),
    scratch_types=[pltpu.VMEM((_CHUNK,), jnp.float32)] * 9
    + [pltpu.VMEM((48,), jnp.float32)]
    + [pltpu.VMEM((_CHUNK,), jnp.float32)] * 6,
)(_sc_transform_body)


def _tc_body(d_r, out_ref, sl_r):
    x1_r = d_r.at[0]
    y1_r = d_r.at[1]
    x2_r = d_r.at[2]
    y2_r = d_r.at[3]
    ar_r = d_r.at[4]
    i2 = (lax.broadcasted_iota(jnp.int32, (_ROWS, 128), 0) * 128
          + lax.broadcasted_iota(jnp.int32, (_ROWS, 128), 1))
    imin = jnp.int32(_IMIN)
    fkey = d_r[5].view(jnp.int32)

    def bs_bit(b, cu):
        cand = cu | lax.shift_left(jnp.int32(1), 31 - b)
        cnt = jnp.sum((fkey >= (cand ^ imin)).astype(jnp.int32))
        return jnp.where(cnt >= _PRE_NMS, cand, cu)

    t_bits = lax.fori_loop(0, 32, bs_bit, jnp.int32(0))
    ft = t_bits ^ imin
    slots = _PRE_NMS - jnp.sum((fkey > ft).astype(jnp.int32))
    eqm = (fkey == ft).astype(jnp.int32)

    def bs_idx(b, c):
        cand = c | lax.shift_left(jnp.int32(1), 16 - b)
        cnt = jnp.sum(jnp.where(i2 < cand, eqm, 0))
        return jnp.where(cnt <= slots, cand, c)

    c = lax.fori_loop(1, 17, bs_idx, jnp.int32(0))
    elig = (fkey > ft) | ((t_bits != 0) & (fkey == ft) & (i2 < c))
    sl_r[...] = jnp.where(elig, fkey, imin)

    lane = lax.broadcasted_iota(jnp.int32, (1, 128), 1)
    zero = jnp.float32(0.0)

    def extract(ref, pick):
        pr = pick // 128
        lm = lane == lax.rem(pick, 128)
        return jnp.sum(jnp.where(lm, ref[pl.ds(pr, 1), :], zero))

    def mkrow(a, b, cc, dd):
        row = jnp.zeros((1, 128), jnp.float32)
        row = jnp.where(lane == 1, a, row)
        row = jnp.where(lane == 2, b, row)
        row = jnp.where(lane == 3, cc, row)
        return jnp.where(lane == 4, dd, row)

    def nms_cond(st):
        k, m = st
        return (k < _POST_NMS) & (m > imin)

    def nms_step(st):
        k, m = st
        sl = sl_r[...]
        pick1 = jnp.min(jnp.where(sl == m, i2, _NP))
        px1 = extract(x1_r, pick1)
        py1 = extract(y1_r, pick1)
        px2 = extract(x2_r, pick1)
        py2 = extract(y2_r, pick1)
        pa = extract(ar_r, pick1)
        sl2 = jnp.where(i2 == pick1, imin, sl)
        m2 = jnp.max(sl2)
        found2 = m2 > imin
        pick2 = jnp.min(jnp.where(sl2 == m2, i2, _NP))
        pick2 = jnp.where(pick2 == _NP, 0, pick2)
        qx1 = extract(x1_r, pick2)
        qy1 = extract(y1_r, pick2)
        qx2 = extract(x2_r, pick2)
        qy2 = extract(y2_r, pick2)
        qa = extract(ar_r, pick2)
        w12 = jnp.maximum(0.0, jnp.minimum(px2, qx2)
                          - jnp.maximum(px1, qx1) + 1.0)
        h12 = jnp.maximum(0.0, jnp.minimum(py2, qy2)
                          - jnp.maximum(py1, qy1) + 1.0)
        i12 = w12 * h12
        o12 = i12 / (pa + qa - i12)
        both = found2 & jnp.logical_not(o12 > _NMS_T)
        ww1 = jnp.maximum(0.0, jnp.minimum(px2, x2_r[...])
                          - jnp.maximum(px1, x1_r[...]) + 1.0)
        hh1 = jnp.maximum(0.0, jnp.minimum(py2, y2_r[...])
                          - jnp.maximum(py1, y1_r[...]) + 1.0)
        in1 = ww1 * hh1
        ov1 = in1 / (pa + ar_r[...] - in1)
        ww2 = jnp.maximum(0.0, jnp.minimum(qx2, x2_r[...])
                          - jnp.maximum(qx1, x1_r[...]) + 1.0)
        hh2 = jnp.maximum(0.0, jnp.minimum(qy2, y2_r[...])
                          - jnp.maximum(qy1, y1_r[...]) + 1.0)
        in2 = ww2 * hh2
        ov2 = in2 / (qa + ar_r[...] - in2)
        kill = (ov1 > _NMS_T) | (both & (ov2 > _NMS_T))
        nsl = jnp.where(kill, imin, sl)
        sl_r[...] = nsl
        out_ref[pl.ds(k, 1), :] = mkrow(px1, py1, px2, py2)
        prev = out_ref[pl.ds(k + 1, 1), :]
        out_ref[pl.ds(k + 1, 1), :] = jnp.where(
            both, mkrow(qx1, qy1, qx2, qy2), prev)
        return (k + 1 + both.astype(jnp.int32), jnp.max(nsl))

    m0 = jnp.max(sl_r[...])
    kf, _ = lax.while_loop(nms_cond, nms_step, (jnp.int32(0), m0))

    e0 = mkrow(jnp.sum(jnp.where(lane == 0, x1_r[0:1, :], zero)),
               jnp.sum(jnp.where(lane == 0, y1_r[0:1, :], zero)),
               jnp.sum(jnp.where(lane == 0, x2_r[0:1, :], zero)),
               jnp.sum(jnp.where(lane == 0, y2_r[0:1, :], zero)))
    fill = jnp.where(kf == 0, e0, out_ref[0:1, :])

    def fill_step(q, _):
        out_ref[pl.ds(q, 1), :] = fill
        return 0

    lax.fori_loop(kf, _POST_NMS, fill_step, 0)


@jax.jit
def kernel(rpn_cls_prob_reshape, rpn_bbox_pred, im_info):
    scores = jnp.transpose(rpn_cls_prob_reshape[:, _A:], (0, 2, 3, 1)).reshape(-1)
    d = jnp.transpose(rpn_bbox_pred, (0, 2, 3, 1)).reshape(-1, 4)
    pad = _NP - _N

    def p1(v):
        return jnp.pad(v, (0, pad))

    imrow = jnp.concatenate([
        jnp.full((16,), im_info[0, 0], jnp.float32),
        jnp.full((16,), im_info[0, 1], jnp.float32),
        jnp.full((16,), _MIN_SIZE * im_info[0, 2], jnp.float32),
    ])
    res = _sc_transform(
        p1(scores), p1(d[:, 0]), p1(d[:, 1]), p1(d[:, 2]), p1(d[:, 3]),
        jnp.asarray(_AG[0]), jnp.asarray(_AG[1]),
        jnp.asarray(_AG[2]), jnp.asarray(_AG[3]), imrow)

    out = pl.pallas_call(
        _tc_body,
        out_shape=jax.ShapeDtypeStruct((_OUT_ROWS, 128), jnp.float32),
        in_specs=[pl.BlockSpec(memory_space=pltpu.VMEM)],
        out_specs=pl.BlockSpec(memory_space=pltpu.VMEM),
        scratch_shapes=[pltpu.VMEM((_ROWS, 128), jnp.int32)],
    )(res.reshape(6, _ROWS, 128))
    return out[:_POST_NMS, :5]
